# Initial kernel scaffold; baseline (speedup 1.0000x reference)
import jax
import jax.numpy as jnp
from jax.experimental import pallas as pl


def kernel(input_ids, table):
    raise NotImplementedError("write your pallas kernel here")


# speedup vs baseline: 3.2340x; 3.2340x over previous
import functools

import jax
import jax.numpy as jnp
from jax import lax
from jax.experimental import pallas as pl
from jax.experimental.pallas import tpu as pltpu
from jax.experimental.pallas import tpu_sc as plsc

BATCH = 1024
SEQ = 200
EMBED_DIM = 64
B_TOTAL = BATCH * SEQ

NC, NS = 2, 16
NW = NC * NS
ROWS_PER_W = B_TOTAL // NW
CHUNK = 128
G = 5
GROUP_ROWS = CHUNK * G
NGROUPS = ROWS_PER_W // GROUP_ROWS
IDX_ROWS_PER_W = ROWS_PER_W // CHUNK

_mesh = plsc.VectorSubcoreMesh(core_axis_name="c", subcore_axis_name="s")


@functools.partial(
    pl.kernel,
    out_type=jax.ShapeDtypeStruct((B_TOTAL, EMBED_DIM), jnp.float32),
    mesh=_mesh,
    compiler_params=pltpu.CompilerParams(use_tc_tiling_on_sc=False),
    scratch_types=[
        pltpu.VMEM((IDX_ROWS_PER_W, CHUNK), jnp.int32),
        pltpu.VMEM((2, GROUP_ROWS, EMBED_DIM), jnp.float32),
        pltpu.SemaphoreType.DMA,
    ],
)
def _gather_kernel(table_hbm, idx_hbm, out_hbm, idx_v, rows_v, sem):
    wid = lax.axis_index("s") * NC + lax.axis_index("c")
    row_base = wid * ROWS_PER_W

    pltpu.sync_copy(idx_hbm.at[wid], idx_v)

    @pl.loop(0, NGROUPS)
    def _group(g):
        buf = rows_v.at[g % 2]
        copies = []
        for j in range(G):
            copies.append(
                pltpu.async_copy(
                    table_hbm.at[idx_v.at[g * G + j]],
                    buf.at[pl.ds(j * CHUNK, CHUNK)],
                    sem,
                )
            )
        for cp in copies:
            cp.wait()
        pltpu.sync_copy(
            buf, out_hbm.at[pl.ds(row_base + g * GROUP_ROWS, GROUP_ROWS)]
        )


def kernel(input_ids, table):
    idx = input_ids.reshape(NW, IDX_ROWS_PER_W, CHUNK).astype(jnp.int32)
    out = _gather_kernel(table, idx)
    return out.reshape(BATCH, SEQ, EMBED_DIM)


# speedup vs baseline: 3.3038x; 1.0216x over previous
import functools

import jax
import jax.numpy as jnp
from jax import lax
from jax.experimental import pallas as pl
from jax.experimental.pallas import tpu as pltpu
from jax.experimental.pallas import tpu_sc as plsc

BATCH = 1024
SEQ = 200
EMBED_DIM = 64
B_TOTAL = BATCH * SEQ

NC, NS = 2, 16
NW = NC * NS
ROWS_PER_W = B_TOTAL // NW
CHUNK = 128
G = 5
GROUP_ROWS = CHUNK * G
NGROUPS = ROWS_PER_W // GROUP_ROWS
IDX_ROWS_PER_W = ROWS_PER_W // CHUNK

_mesh = plsc.VectorSubcoreMesh(core_axis_name="c", subcore_axis_name="s")


@functools.partial(
    pl.kernel,
    out_type=jax.ShapeDtypeStruct((B_TOTAL, EMBED_DIM), jnp.float32),
    mesh=_mesh,
    compiler_params=pltpu.CompilerParams(use_tc_tiling_on_sc=False),
    scratch_types=[
        pltpu.VMEM((IDX_ROWS_PER_W, CHUNK), jnp.int32),
        pltpu.VMEM((2, GROUP_ROWS, EMBED_DIM), jnp.float32),
        pltpu.SemaphoreType.DMA,
        pltpu.SemaphoreType.DMA,
    ],
)
def _gather_kernel(table_hbm, idx_hbm, out_hbm, idx_v, rows_v, gsem, osem):
    wid = lax.axis_index("s") * NC + lax.axis_index("c")
    row_base = wid * ROWS_PER_W

    pltpu.sync_copy(idx_hbm.at[wid], idx_v)

    def _fire(g):
        buf = rows_v.at[g % 2]
        for j in range(G):
            pltpu.async_copy(
                table_hbm.at[idx_v.at[g * G + j]],
                buf.at[pl.ds(j * CHUNK, CHUNK)],
                gsem,
            )

    def _out_slice(g):
        return out_hbm.at[pl.ds(row_base + g * GROUP_ROWS, GROUP_ROWS)]

    _fire(0)

    @pl.loop(0, NGROUPS)
    def _group(g):
        @pl.when(g >= 1)
        def _():
            pltpu.make_async_copy(
                rows_v.at[(g + 1) % 2], _out_slice(g - 1), osem
            ).wait()

        @pl.when(g + 1 < NGROUPS)
        def _():
            _fire(g + 1)

        buf = rows_v.at[g % 2]
        for j in range(G):
            pltpu.make_async_copy(
                table_hbm.at[idx_v.at[g * G + j]],
                buf.at[pl.ds(j * CHUNK, CHUNK)],
                gsem,
            ).wait()
        pltpu.async_copy(buf, _out_slice(g), osem)

    pltpu.make_async_copy(
        rows_v.at[(NGROUPS - 1) % 2], _out_slice(NGROUPS - 1), osem
    ).wait()


def kernel(input_ids, table):
    idx = input_ids.reshape(NW, IDX_ROWS_PER_W, CHUNK).astype(jnp.int32)
    out = _gather_kernel(table, idx)
    return out.reshape(BATCH, SEQ, EMBED_DIM)
